# Initial kernel scaffold; baseline (speedup 1.0000x reference)
import jax
import jax.numpy as jnp
from jax.experimental import pallas as pl


def kernel(position, elements, polynomials, lam, mu, measure):
    raise NotImplementedError("write your pallas kernel here")


# speedup vs baseline: 11.0966x; 11.0966x over previous
import jax
import jax.numpy as jnp
from jax import lax
from jax.experimental import pallas as pl
from jax.experimental.pallas import tpu as pltpu
from jax.experimental.pallas import tpu_sc as plsc

N_VERT = 10000
N_ELEM = 160000
NC, NS, L = 2, 16, 16
NW = NC * NS
PER_W = N_ELEM // NW
CHUNK = 1000
N_CHUNKS = PER_W // CHUNK
STEPS = (CHUNK + L - 1) // L

_LN2 = 0.6931471805599453
_SQRT2 = 1.4142135623730951


def _vlog(x):
    bits = lax.bitcast_convert_type(x, jnp.int32)
    e = jnp.right_shift(bits, 23) - 127
    m_bits = jnp.bitwise_or(jnp.bitwise_and(bits, 0x007FFFFF), 0x3F800000)
    m = lax.bitcast_convert_type(m_bits, jnp.float32)
    big = m > _SQRT2
    m = jnp.where(big, m * 0.5, m)
    e = jnp.where(big, e + 1, e)
    t = (m - 1.0) / (m + 1.0)
    t2 = t * t
    p = 2.0 * t * (1.0 + t2 * (1.0 / 3.0 + t2 * (1.0 / 5.0 + t2 * (1.0 / 7.0))))
    return e.astype(jnp.float32) * _LN2 + p


def _body(pos_hbm, elem_hbm, poly_hbm, lam_hbm, mu_hbm, meas_hbm, out_hbm,
          pos_v, elem_v, poly_v, lam_v, mu_v, meas_v, acc_v):
    c = lax.axis_index("c")
    s = lax.axis_index("s")
    wid = s * NC + c

    pltpu.sync_copy(pos_hbm, pos_v)
    lanes = lax.broadcasted_iota(jnp.int32, (L,), 0)

    def chunk_body(ci, acc):
        base = wid * PER_W + ci * CHUNK
        pltpu.sync_copy(elem_hbm.at[pl.ds(base * 4, CHUNK * 4)], elem_v)
        pltpu.sync_copy(poly_hbm.at[pl.ds(base * 16, CHUNK * 16)], poly_v)
        pltpu.sync_copy(lam_hbm.at[pl.ds(base, CHUNK)], lam_v.at[pl.ds(0, CHUNK)])
        pltpu.sync_copy(mu_hbm.at[pl.ds(base, CHUNK)], mu_v.at[pl.ds(0, CHUNK)])
        pltpu.sync_copy(meas_hbm.at[pl.ds(base, CHUNK)], meas_v.at[pl.ds(0, CHUNK)])

        def step(si, acc):
            o = si * L + lanes
            valid = o < CHUNK
            oc = jnp.minimum(o, CHUNK - 1)
            oc4 = oc * 4
            oc16 = oc * 16
            ev = [plsc.load_gather(elem_v, [oc4 + f]) for f in range(4)]
            ev3 = [jnp.minimum(jnp.maximum(e, 0), N_VERT - 1) * 3 for e in ev]
            p = [[plsc.load_gather(pos_v, [ev3[f] + t]) for t in range(3)]
                 for f in range(4)]
            b = [[plsc.load_gather(poly_v, [oc16 + (4 * f + l)])
                  for l in range(3)] for f in range(4)]
            F = [[p[0][t] * b[0][l] + p[1][t] * b[1][l]
                  + p[2][t] * b[2][l] + p[3][t] * b[3][l]
                  for l in range(3)] for t in range(3)]
            Ic = (F[0][0] * F[0][0] + F[0][1] * F[0][1] + F[0][2] * F[0][2]
                  + F[1][0] * F[1][0] + F[1][1] * F[1][1] + F[1][2] * F[1][2]
                  + F[2][0] * F[2][0] + F[2][1] * F[2][1] + F[2][2] * F[2][2])
            J = (F[0][0] * (F[1][1] * F[2][2] - F[1][2] * F[2][1])
                 - F[0][1] * (F[1][0] * F[2][2] - F[1][2] * F[2][0])
                 + F[0][2] * (F[1][0] * F[2][1] - F[1][1] * F[2][0]))
            lam = lam_v[pl.ds(si * L, L)]
            mu = mu_v[pl.ds(si * L, L)]
            meas = meas_v[pl.ds(si * L, L)]
            alpha = 0.75 * mu / lam + 1.0
            ic_v = jnp.maximum(Ic + 1.0, 0.0) + 1e-30
            d = J - alpha
            psi = 0.5 * mu * (Ic - 3.0) + 0.5 * lam * d * d - 0.5 * mu * _vlog(ic_v)
            return acc + jnp.where(valid, psi * meas, 0.0)

        return lax.fori_loop(0, STEPS, step, acc)

    acc = lax.fori_loop(0, N_CHUNKS, chunk_body, jnp.zeros((L,), jnp.float32))
    acc_v[...] = acc
    pltpu.sync_copy(acc_v, out_hbm.at[pl.ds(wid * L, L)])


@jax.jit
def kernel(position, elements, polynomials, lam, mu, measure):
    mesh = plsc.VectorSubcoreMesh(core_axis_name="c", subcore_axis_name="s",
                                  num_cores=NC, num_subcores=NS)
    partials = pl.kernel(
        _body,
        out_type=jax.ShapeDtypeStruct((NW * L,), jnp.float32),
        mesh=mesh,
        compiler_params=pltpu.CompilerParams(needs_layout_passes=False),
        scratch_types=[
            pltpu.VMEM((N_VERT * 3,), jnp.float32),
            pltpu.VMEM((CHUNK * 4,), jnp.int32),
            pltpu.VMEM((CHUNK * 16,), jnp.float32),
            pltpu.VMEM((CHUNK + 24,), jnp.float32),
            pltpu.VMEM((CHUNK + 24,), jnp.float32),
            pltpu.VMEM((CHUNK + 24,), jnp.float32),
            pltpu.VMEM((L,), jnp.float32),
        ],
    )(position.reshape(-1), elements.reshape(-1), polynomials.reshape(-1),
      lam, mu, measure)
    return jnp.sum(partials)
